# baseline (speedup 1.0000x reference)
import functools

import jax
import jax.numpy as jnp
from jax import lax
from jax.experimental import pallas as pl
from jax.experimental.pallas import tpu as pltpu
from jax.experimental.pallas import tpu_sc as plsc

CYCLE = 168
L_OUT = 200
CH = 64
NC = 2
NS = 16
NW = NC * NS
LANES = 16
NREP = 8
TBL = CH * CYCLE


def _sc_transposed_windows(base_idx, data):
    B = base_idx.shape[0]
    b_per_w = B // NW
    n_vregs = b_per_w // LANES

    mesh = plsc.VectorSubcoreMesh(core_axis_name="c", subcore_axis_name="s")

    @functools.partial(
        pl.kernel,
        out_type=jax.ShapeDtypeStruct((L_OUT, CH, B), jnp.float32),
        mesh=mesh,
        compiler_params=pltpu.CompilerParams(needs_layout_passes=False),
        scratch_types=[
            pltpu.VMEM((b_per_w,), jnp.int32),
            pltpu.VMEM((NREP, TBL + 1), jnp.float32),
            pltpu.VMEM((CH, b_per_w), jnp.float32),
            pltpu.VMEM((CH, b_per_w), jnp.float32),
            pltpu.SemaphoreType.DMA,
            pltpu.SemaphoreType.DMA,
        ],
    )
    def k(idx_hbm, data_hbm, out_hbm, idx_v, table_v, buf_a, buf_b,
          sem_a, sem_b):
        wid = lax.axis_index("s") * NC + lax.axis_index("c")
        b0 = wid * b_per_w
        for r in range(NREP):
            pltpu.sync_copy(data_hbm, table_v.at[r, pl.ds(0, TBL)])
        pltpu.sync_copy(idx_hbm.at[pl.ds(b0, b_per_w)], idx_v)
        lane = lax.broadcasted_iota(jnp.int32, (LANES,), 0)
        rep = lane & (NREP - 1)

        iv0 = tuple(idx_v[pl.ds(k * LANES, LANES)] for k in range(n_vregs))
        bufs = (buf_a, buf_b)
        sems = (sem_a, sem_b)

        def out_block(l):
            return out_hbm.at[l, :, pl.ds(b0, b_per_w)]

        def step(q, l, iv, buf, sem):
            @pl.when(q >= 1)
            def _wait_prev(l=l):
                pltpu.make_async_copy(buf, out_block(l - 2), sem).wait()

            @pl.when(jnp.logical_and(q >= 1, l - 2 < L_OUT - CYCLE))
            def _wait_prev_dup(l=l):
                pltpu.make_async_copy(buf, out_block(l - 2), sem).wait()

            for c0 in range(0, CH, 2):
                vals = []
                for cc in (0, 1):
                    cb = (c0 + cc) * CYCLE
                    for g in range(n_vregs):
                        vals.append(
                            plsc.load_gather(table_v, [rep, iv[g] + cb]))
                i = 0
                for cc in (0, 1):
                    for g in range(n_vregs):
                        buf[c0 + cc, pl.ds(g * LANES, LANES)] = vals[i]
                        i += 1
            pltpu.async_copy(buf, out_block(l), sem)

            @pl.when(l < L_OUT - CYCLE)
            def _dup(l=l):
                pltpu.async_copy(buf, out_block(l + CYCLE), sem)

            nxt = []
            for g in range(n_vregs):
                v = iv[g] + 1
                nxt.append(jnp.where(v == CYCLE, 0, v))
            return tuple(nxt)

        def pair(q, iv):
            for p in (0, 1):
                iv = step(q, 2 * q + p, iv, bufs[p], sems[p])
            return iv

        lax.fori_loop(0, CYCLE // 2, pair, iv0)
        for p in (0, 1):
            pltpu.make_async_copy(
                bufs[p], out_block(CYCLE - 2 + p), sems[p]).wait()

    return k(base_idx, data)


def kernel(index, length, data):
    B = index.shape[0]
    base_idx = ((index.reshape(B).astype(jnp.int32) + (length - L_OUT))
                % CYCLE).astype(jnp.int32)
    out_t = _sc_transposed_windows(base_idx, data.T.reshape(TBL))
    return jnp.transpose(out_t, (2, 0, 1))


# speedup vs baseline: 1.0833x; 1.0833x over previous
import functools

import jax
import jax.numpy as jnp
from jax import lax
from jax.experimental import pallas as pl
from jax.experimental.pallas import tpu as pltpu
from jax.experimental.pallas import tpu_sc as plsc

CYCLE = 168
L_OUT = 200
CH = 64
NC = 2
NS = 16
NW = NC * NS
LANES = 16


def _sc_transposed_windows(base_idx, data):
    B = base_idx.shape[0]
    b_per_w = B // NW
    n_vregs = b_per_w // LANES

    mesh = plsc.VectorSubcoreMesh(core_axis_name="c", subcore_axis_name="s")

    @functools.partial(
        pl.kernel,
        out_type=jax.ShapeDtypeStruct((L_OUT, CH, B), jnp.float32),
        mesh=mesh,
        compiler_params=pltpu.CompilerParams(needs_layout_passes=False),
        scratch_types=[
            pltpu.VMEM((b_per_w,), jnp.int32),
            pltpu.VMEM((CH, CYCLE), jnp.float32),
            pltpu.VMEM((CH, b_per_w), jnp.float32),
            pltpu.VMEM((CH, b_per_w), jnp.float32),
            pltpu.SemaphoreType.DMA,
            pltpu.SemaphoreType.DMA,
        ],
    )
    def k(idx_hbm, data_hbm, out_hbm, idx_v, table_v, buf_a, buf_b,
          sem_a, sem_b):
        wid = lax.axis_index("s") * NC + lax.axis_index("c")
        b0 = wid * b_per_w
        pltpu.sync_copy(data_hbm, table_v)
        pltpu.sync_copy(idx_hbm.at[pl.ds(b0, b_per_w)], idx_v)

        iv0 = tuple(idx_v[pl.ds(k * LANES, LANES)] for k in range(n_vregs))
        bufs = (buf_a, buf_b)
        sems = (sem_a, sem_b)

        def out_block(l):
            return out_hbm.at[l, :, pl.ds(b0, b_per_w)]

        def step(q, l, iv, buf, sem):
            @pl.when(q >= 1)
            def _wait_prev(l=l):
                pltpu.make_async_copy(buf, out_block(l - 2), sem).wait()

            @pl.when(jnp.logical_and(q >= 1, l - 2 < L_OUT - CYCLE))
            def _wait_prev_dup(l=l):
                pltpu.make_async_copy(buf, out_block(l - 2), sem).wait()

            for c0 in range(0, CH, 2):
                vals = []
                for cc in (0, 1):
                    cs = jnp.full((LANES,), c0 + cc, jnp.int32)
                    for g in range(n_vregs):
                        vals.append(plsc.load_gather(table_v, [cs, iv[g]]))
                i = 0
                for cc in (0, 1):
                    for g in range(n_vregs):
                        buf[c0 + cc, pl.ds(g * LANES, LANES)] = vals[i]
                        i += 1
            pltpu.async_copy(buf, out_block(l), sem)

            @pl.when(l < L_OUT - CYCLE)
            def _dup(l=l):
                pltpu.async_copy(buf, out_block(l + CYCLE), sem)

            nxt = []
            for g in range(n_vregs):
                v = iv[g] + 1
                nxt.append(jnp.where(v == CYCLE, 0, v))
            return tuple(nxt)

        def pair(q, iv):
            for p in (0, 1):
                iv = step(q, 2 * q + p, iv, bufs[p], sems[p])
            return iv

        lax.fori_loop(0, CYCLE // 2, pair, iv0)
        for p in (0, 1):
            pltpu.make_async_copy(
                bufs[p], out_block(CYCLE - 2 + p), sems[p]).wait()

    return k(base_idx, data)


def kernel(index, length, data):
    B = index.shape[0]
    base_idx = ((index.reshape(B).astype(jnp.int32) + (length - L_OUT))
                % CYCLE).astype(jnp.int32)
    out_t = _sc_transposed_windows(base_idx, data.T)
    return jnp.transpose(out_t, (2, 0, 1))
